# Initial kernel scaffold; baseline (speedup 1.0000x reference)
import jax
import jax.numpy as jnp
from jax.experimental import pallas as pl


def kernel(features, edge_index, edge_types, V1, comp1, loop_w1, b1, V2, comp2, loop_w2, b2, pred_w, pred_b):
    raise NotImplementedError("write your pallas kernel here")


# speedup vs baseline: 21.5742x; 21.5742x over previous
import functools

import jax
import jax.numpy as jnp
from jax import lax
from jax.experimental import pallas as pl
from jax.experimental.pallas import tpu as pltpu
from jax.experimental.pallas import tpu_sc as plsc

_NC = 2
_NS = 16
_NW = _NC * _NS
_CH = 80


def _prep(x, V, comp, loop_w, b):
    N, D = x.shape
    BLK = 1000
    nb = N // BLK

    def body(comp_ref, x_ref, V_ref, loop_ref, b_ref, out_ref):
        r = pl.program_id(1)
        rc = jnp.minimum(r, 3)
        c0 = comp_ref[rc, 0]
        c1 = comp_ref[rc, 1]
        Wm = c0 * V_ref[0] + c1 * V_ref[1]
        Wm = jnp.where(r < 4, Wm, loop_ref[...])
        y = jnp.dot(x_ref[...], Wm, preferred_element_type=jnp.float32)
        y = y + jnp.where(r == 4, 1.0, 0.0) * b_ref[...]
        y = jnp.where(r == 5, jnp.zeros_like(y), y)
        out_ref[...] = y[None]

    return pl.pallas_call(
        body,
        grid=(nb, 6),
        in_specs=[
            pl.BlockSpec(memory_space=pltpu.SMEM),
            pl.BlockSpec((BLK, D), lambda bb, rr: (bb, 0)),
            pl.BlockSpec((2, D, D), lambda bb, rr: (0, 0, 0)),
            pl.BlockSpec((D, D), lambda bb, rr: (0, 0)),
            pl.BlockSpec((1, D), lambda bb, rr: (0, 0)),
        ],
        out_specs=pl.BlockSpec((1, BLK, D), lambda bb, rr: (rr, bb, 0)),
        out_shape=jax.ShapeDtypeStruct((6, N, D), jnp.float32),
    )(comp, x, V, loop_w, b.reshape(1, D))


def _gather_idx(edge_types, src, N):
    E = src.shape[0]
    rows = E // 128
    et2 = edge_types.reshape(rows, 128)
    s2 = src.reshape(rows, 128)

    def body(a_ref, b_ref, o_ref):
        o_ref[...] = a_ref[...] * N + b_ref[...]

    return pl.pallas_call(
        body,
        out_shape=jax.ShapeDtypeStruct((rows, 128), jnp.int32),
    )(et2, s2)


def _combine(parts):
    _, N, D = parts.shape
    BLK = 1000
    nb = N // BLK

    def body(p_ref, o_ref):
        o_ref[...] = jnp.maximum(p_ref[0] + p_ref[1], 0.0)

    return pl.pallas_call(
        body,
        grid=(nb,),
        in_specs=[pl.BlockSpec((2, BLK, D), lambda bb: (0, bb, 0))],
        out_specs=pl.BlockSpec((BLK, D), lambda bb: (bb, 0)),
        out_shape=jax.ShapeDtypeStruct((N, D), jnp.float32),
    )(parts)


def _final(parts, pred_w, pred_b):
    _, N, D = parts.shape
    BLK = 1000
    nb = N // BLK

    def body(pb_ref, p_ref, pw_ref, o_ref):
        h = jnp.maximum(p_ref[0] + p_ref[1], 0.0)
        lg = jnp.sum(h * pw_ref[...], axis=1, keepdims=True) + pb_ref[0]
        o_ref[...] = jnp.broadcast_to(jax.nn.sigmoid(lg), (BLK, D))

    return pl.pallas_call(
        body,
        grid=(nb,),
        in_specs=[
            pl.BlockSpec(memory_space=pltpu.SMEM),
            pl.BlockSpec((2, BLK, D), lambda bb: (0, bb, 0)),
            pl.BlockSpec((1, D), lambda bb: (0, 0)),
        ],
        out_specs=pl.BlockSpec((BLK, D), lambda bb: (bb, 0)),
        out_shape=jax.ShapeDtypeStruct((N, D), jnp.float32),
    )(pred_b, parts, pred_w.reshape(1, D))


def _sc_layer(table_flat, idx_r, dst_r, N, D):
    NCH, CH = idx_r.shape[1], idx_r.shape[2]
    NBLK8 = N // 8
    base_blocks = NBLK8 // _NS
    extra = NBLK8 - base_blocks * _NS
    MAIN = base_blocks * 8

    mesh = plsc.VectorSubcoreMesh(
        core_axis_name="c", subcore_axis_name="s",
        num_cores=_NC, num_subcores=_NS)

    @functools.partial(
        pl.kernel,
        out_type=jax.ShapeDtypeStruct((_NC, N, D), jnp.float32),
        mesh=mesh,
        scratch_types=[
            pltpu.VMEM((NCH, CH), jnp.int32),
            pltpu.VMEM((NCH, CH), jnp.int32),
            pltpu.VMEM((CH, D), jnp.float32),
            pltpu.VMEM_SHARED((N, D), jnp.float32),
            pltpu.SemaphoreType.DMA,
        ],
    )
    def k(table_hbm, idx_hbm, dst_hbm, out_hbm, idx_v, dst_v, rows_v, acc, sem):
        c = lax.axis_index("c")
        s = lax.axis_index("s")
        w = c * _NS + s
        row0 = 8 * (s * base_blocks + jnp.minimum(s, extra))
        tbase = (4 + c) * N
        pltpu.sync_copy(table_hbm.at[pl.ds(tbase + row0, MAIN), :],
                        acc.at[pl.ds(row0, MAIN), :])

        @pl.when(s < extra)
        def _():
            pltpu.sync_copy(table_hbm.at[pl.ds(tbase + row0 + MAIN, 8), :],
                            acc.at[pl.ds(row0 + MAIN, 8), :])

        pltpu.sync_copy(idx_hbm.at[w], idx_v)
        pltpu.sync_copy(dst_hbm.at[w], dst_v)
        plsc.subcore_barrier()

        def body(i, carry):
            pltpu.async_copy(table_hbm.at[idx_v.at[i]], rows_v, sem).wait()
            pltpu.sync_copy(rows_v, acc.at[dst_v.at[i]], add=True)
            return carry

        lax.fori_loop(0, NCH, body, 0)
        plsc.subcore_barrier()
        pltpu.sync_copy(acc.at[pl.ds(row0, MAIN), :],
                        out_hbm.at[c, pl.ds(row0, MAIN), :])

        @pl.when(s < extra)
        def _():
            pltpu.sync_copy(acc.at[pl.ds(row0 + MAIN, 8), :],
                            out_hbm.at[c, pl.ds(row0 + MAIN, 8), :])

    return k(table_flat, idx_r, dst_r)


def kernel(features, edge_index, edge_types, V1, comp1, loop_w1, b1,
           V2, comp2, loop_w2, b2, pred_w, pred_b):
    N, D = features.shape
    E = edge_index.shape[1]
    src = edge_index[0]
    dst = edge_index[1]

    epw = E // _NW
    nch = epw // _CH

    idx = _gather_idx(edge_types, src, N)
    idx_r = idx.reshape(_NW, nch, _CH)
    dst_r = dst.reshape(_NW, nch, _CH)

    table1 = _prep(features, V1, comp1, loop_w1, b1).reshape(6 * N, D)
    parts1 = _sc_layer(table1, idx_r, dst_r, N, D)
    h1 = _combine(parts1)

    table2 = _prep(h1, V2, comp2, loop_w2, b2).reshape(6 * N, D)
    parts2 = _sc_layer(table2, idx_r, dst_r, N, D)

    out = _final(parts2, pred_w, pred_b)
    return out[:, 0]
